# Initial kernel scaffold; baseline (speedup 1.0000x reference)
import jax
import jax.numpy as jnp
from jax.experimental import pallas as pl


def kernel(z_vals, bin_edges, table):
    raise NotImplementedError("write your pallas kernel here")


# speedup vs baseline: 7.8664x; 7.8664x over previous
import jax
import jax.numpy as jnp
from jax import lax
from jax.experimental import pallas as pl
from jax.experimental.pallas import tpu as pltpu
from jax.experimental.pallas import tpu_sc as plsc

NUM_HEADS = 8
NUM_BINS = 16
N_EDGES_INNER = NUM_BINS - 1
E_TOTAL = 3200000
NC, NS, L = 2, 16, 16
NW = NC * NS
PER_W = E_TOTAL // NW
CHUNK = 2000
NCHUNK = PER_W // CHUNK
GROUPS = CHUNK // L
UNROLL = 5


def _body(z_hbm, edges_hbm, tablet_hbm, out_hbm, zbuf, obuf, table_v, edges_v):
    wid = lax.axis_index("s") * NC + lax.axis_index("c")
    wbase = wid * PER_W

    pltpu.sync_copy(tablet_hbm, table_v)
    pltpu.sync_copy(edges_hbm, edges_v)

    edges = [
        plsc.load_gather(edges_v, [jnp.full((L,), i + 1, jnp.int32)])
        for i in range(N_EDGES_INNER)
    ]
    iota = lax.iota(jnp.int32, L)
    gpat = [jnp.full((L,), h * NUM_BINS, jnp.int32) for h in range(NUM_HEADS)]
    spat = [iota * NUM_HEADS + h for h in range(NUM_HEADS)]

    def chunk_body(c, _):
        zoff = wbase + c * CHUNK
        pltpu.sync_copy(z_hbm.at[pl.ds(zoff, CHUNK)], zbuf)

        def grp_body(gi, _):
            for u in range(UNROLL):
                off = (gi * UNROLL + u) * L
                z = zbuf[pl.ds(off, L)]
                bin_idx = jnp.zeros((L,), jnp.int32)
                one = jnp.ones((L,), jnp.int32)
                zero = jnp.zeros((L,), jnp.int32)
                for e in edges:
                    bin_idx += jnp.where(z > e, one, zero)
                obase = off * NUM_HEADS
                for h in range(NUM_HEADS):
                    vals = plsc.load_gather(table_v, [bin_idx + gpat[h]])
                    plsc.store_scatter(obuf, [spat[h] + obase], vals)
            return 0

        lax.fori_loop(0, GROUPS // UNROLL, grp_body, 0)
        pltpu.sync_copy(obuf, out_hbm.at[pl.ds(zoff * NUM_HEADS, CHUNK * NUM_HEADS)])
        return 0

    lax.fori_loop(0, NCHUNK, chunk_body, 0)


def kernel(z_vals, bin_edges, table):
    edges16 = jnp.pad(bin_edges[1:NUM_BINS], (1, 0))
    tablet = table.T.reshape(-1)

    mesh = plsc.VectorSubcoreMesh(core_axis_name="c", subcore_axis_name="s")
    out = pl.kernel(
        _body,
        out_type=jax.ShapeDtypeStruct((E_TOTAL * NUM_HEADS,), jnp.float32),
        mesh=mesh,
        compiler_params=pltpu.CompilerParams(needs_layout_passes=False),
        scratch_types=[
            pltpu.VMEM((CHUNK,), jnp.float32),
            pltpu.VMEM((CHUNK * NUM_HEADS,), jnp.float32),
            pltpu.VMEM((NUM_BINS * NUM_HEADS,), jnp.float32),
            pltpu.VMEM((L,), jnp.float32),
        ],
    )(z_vals, edges16, tablet)
    return out.reshape(E_TOTAL, NUM_HEADS)


# speedup vs baseline: 7.8762x; 1.0012x over previous
import jax
import jax.numpy as jnp
from jax import lax
from jax.experimental import pallas as pl
from jax.experimental.pallas import tpu as pltpu
from jax.experimental.pallas import tpu_sc as plsc

NUM_HEADS = 8
NUM_BINS = 16
N_EDGES_INNER = NUM_BINS - 1
E_TOTAL = 3200000
NC, NS, L = 2, 16, 16
NW = NC * NS
PER_W = E_TOTAL // NW
CHUNK = 2000
NCHUNK = PER_W // CHUNK
GROUPS = CHUNK // L
UNROLL = 5


def _body(z_hbm, edges_hbm, tablet_hbm, out_hbm, zbuf, obuf, table_v, edges_v):
    wid = lax.axis_index("s") * NC + lax.axis_index("c")
    wbase = wid * PER_W

    pltpu.sync_copy(tablet_hbm, table_v)
    pltpu.sync_copy(edges_hbm, edges_v)

    edges = [
        plsc.load_gather(edges_v, [jnp.full((L,), i + 1, jnp.int32)])
        for i in range(N_EDGES_INNER)
    ]
    iota = lax.iota(jnp.int32, L)
    gpat = [jnp.full((L,), h * NUM_BINS, jnp.int32) for h in range(NUM_HEADS)]
    spat = [iota * NUM_HEADS + h for h in range(NUM_HEADS)]

    def chunk_body(c, _):
        zoff = wbase + c * CHUNK
        pltpu.sync_copy(z_hbm.at[pl.ds(zoff, CHUNK)], zbuf)

        def grp_body(gi, _):
            for u in range(UNROLL):
                off = (gi * UNROLL + u) * L
                z = zbuf[pl.ds(off, L)]
                bin_idx = jnp.zeros((L,), jnp.int32)
                one = jnp.ones((L,), jnp.int32)
                zero = jnp.zeros((L,), jnp.int32)
                for e in edges:
                    bin_idx += jnp.where(z > e, one, zero)
                obase = off * NUM_HEADS
                for h in range(NUM_HEADS):
                    vals = plsc.load_gather(table_v, [bin_idx + gpat[h]])
                    plsc.store_scatter(obuf, [spat[h] + obase], vals)
            return 0

        lax.fori_loop(0, GROUPS // UNROLL, grp_body, 0)
        pltpu.sync_copy(obuf, out_hbm.at[pl.ds(zoff * NUM_HEADS, CHUNK * NUM_HEADS)])
        return 0

    lax.fori_loop(0, NCHUNK, chunk_body, 0)


def kernel(z_vals, bin_edges, table):
    edges16 = jnp.pad(bin_edges[1:NUM_BINS], (1, 0))
    tablet = table.T.reshape(-1)

    mesh = plsc.VectorSubcoreMesh(core_axis_name="c", subcore_axis_name="s")
    out = pl.kernel(
        _body,
        out_type=jax.ShapeDtypeStruct((E_TOTAL * NUM_HEADS,), jnp.float32),
        mesh=mesh,
        compiler_params=pltpu.CompilerParams(
            needs_layout_passes=False, use_tc_tiling_on_sc=True
        ),
        scratch_types=[
            pltpu.VMEM((CHUNK,), jnp.float32),
            pltpu.VMEM((CHUNK * NUM_HEADS,), jnp.float32),
            pltpu.VMEM((NUM_BINS * NUM_HEADS,), jnp.float32),
            pltpu.VMEM((L,), jnp.float32),
        ],
    )(z_vals, edges16, tablet)
    return out.reshape(E_TOTAL, NUM_HEADS)


# speedup vs baseline: 7.9046x; 1.0036x over previous
import jax
import jax.numpy as jnp
from jax import lax
from jax.experimental import pallas as pl
from jax.experimental.pallas import tpu as pltpu
from jax.experimental.pallas import tpu_sc as plsc

NUM_HEADS = 8
NUM_BINS = 16
N_EDGES_INNER = NUM_BINS - 1
E_TOTAL = 3200000
NC, NS, L = 2, 16, 16
NW = NC * NS
CHUNK = 3200
NCHUNKS = E_TOTAL // CHUNK
ROUNDS = -(-NCHUNKS // NW)
GROUPS = CHUNK // L
UNROLL = 5
OROWS = CHUNK * NUM_HEADS // 128


def _body(z_hbm, edges_hbm, tablet_hbm, out_hbm, zbuf, obuf, table_v, edges_v):
    wid = lax.axis_index("s") * NC + lax.axis_index("c")

    pltpu.sync_copy(tablet_hbm, table_v)
    pltpu.sync_copy(edges_hbm, edges_v)

    edges = [
        plsc.load_gather(edges_v, [jnp.full((L,), i + 1, jnp.int32)])
        for i in range(N_EDGES_INNER)
    ]
    iota = lax.iota(jnp.int32, L)
    gpat = [jnp.full((L,), h * NUM_BINS, jnp.int32) for h in range(NUM_HEADS)]
    spat = [iota * NUM_HEADS + h for h in range(NUM_HEADS)]

    def chunk_body(r, _):
        t = r * NW + wid

        @pl.when(t < NCHUNKS)
        def _():
            zoff = t * CHUNK
            pltpu.sync_copy(z_hbm.at[pl.ds(zoff, CHUNK)], zbuf)

            def grp_body(gi, _):
                for u in range(UNROLL):
                    off = (gi * UNROLL + u) * L
                    z = zbuf[pl.ds(off, L)]
                    bin_idx = jnp.zeros((L,), jnp.int32)
                    one = jnp.ones((L,), jnp.int32)
                    zero = jnp.zeros((L,), jnp.int32)
                    for e in edges:
                        bin_idx += jnp.where(z > e, one, zero)
                    obase = off * NUM_HEADS
                    for h in range(NUM_HEADS):
                        vals = plsc.load_gather(table_v, [bin_idx + gpat[h]])
                        fidx = spat[h] + obase
                        plsc.store_scatter(
                            obuf,
                            [
                                lax.shift_right_logical(fidx, 7),
                                lax.bitwise_and(fidx, 127),
                            ],
                            vals,
                        )
                return 0

            lax.fori_loop(0, GROUPS // UNROLL, grp_body, 0)
            pltpu.sync_copy(obuf, out_hbm.at[pl.ds(t * OROWS, OROWS), :])

        return 0

    lax.fori_loop(0, ROUNDS, chunk_body, 0)


def kernel(z_vals, bin_edges, table):
    edges16 = jnp.pad(bin_edges[1:NUM_BINS], (1, 0))
    tablet = table.T.reshape(-1)

    mesh = plsc.VectorSubcoreMesh(core_axis_name="c", subcore_axis_name="s")
    out = pl.kernel(
        _body,
        out_type=jax.ShapeDtypeStruct((E_TOTAL * NUM_HEADS // 128, 128), jnp.float32),
        mesh=mesh,
        compiler_params=pltpu.CompilerParams(needs_layout_passes=False),
        scratch_types=[
            pltpu.VMEM((CHUNK,), jnp.float32),
            pltpu.VMEM((OROWS, 128), jnp.float32),
            pltpu.VMEM((NUM_BINS * NUM_HEADS,), jnp.float32),
            pltpu.VMEM((L,), jnp.float32),
        ],
    )(z_vals, edges16, tablet)
    return out.reshape(E_TOTAL, NUM_HEADS)


# speedup vs baseline: 7.9061x; 1.0002x over previous
import jax
import jax.numpy as jnp
from jax import lax
from jax.experimental import pallas as pl
from jax.experimental.pallas import tpu as pltpu
from jax.experimental.pallas import tpu_sc as plsc

NUM_HEADS = 8
NUM_BINS = 16
N_EDGES_INNER = NUM_BINS - 1
E_TOTAL = 3200000
NC, NS, L = 2, 16, 16
NW = NC * NS
CHUNK = 3200
NCHUNKS = E_TOTAL // CHUNK
ROUNDS = -(-NCHUNKS // NW)
GROUPS = CHUNK // L
UNROLL = 5
OROWS = CHUNK * NUM_HEADS // 128


def _body(z_hbm, edges_hbm, tablet_hbm, out_hbm, zbuf, obuf, table_v, edges_v):
    wid = lax.axis_index("s") * NC + lax.axis_index("c")

    pltpu.sync_copy(tablet_hbm, table_v)
    pltpu.sync_copy(edges_hbm, edges_v)

    edges = [
        plsc.load_gather(edges_v, [jnp.full((L,), i + 1, jnp.int32)])
        for i in range(N_EDGES_INNER)
    ]
    iota = lax.iota(jnp.int32, L)
    gpat = [jnp.full((L,), h * NUM_BINS, jnp.int32) for h in range(NUM_HEADS)]
    spat = [iota * NUM_HEADS + h for h in range(NUM_HEADS)]

    def chunk_body(r, _):
        t = r * NW + wid

        @pl.when(t < NCHUNKS)
        def _():
            zoff = t * CHUNK
            pltpu.sync_copy(z_hbm.at[pl.ds(zoff, CHUNK)], zbuf)

            def grp_body(gi, _):
                for u in range(UNROLL):
                    off = (gi * UNROLL + u) * L
                    z = zbuf[pl.ds(off, L)]
                    bin_idx = jnp.zeros((L,), jnp.int32)
                    one = jnp.ones((L,), jnp.int32)
                    zero = jnp.zeros((L,), jnp.int32)
                    for e in edges:
                        bin_idx += jnp.where(z > e, one, zero)
                    obase = off * NUM_HEADS
                    for h in range(NUM_HEADS):
                        vals = plsc.load_gather(table_v, [bin_idx + gpat[h]])
                        fidx = spat[h] + obase
                        plsc.store_scatter(
                            obuf,
                            [
                                lax.shift_right_logical(fidx, 7),
                                lax.bitwise_and(fidx, 127),
                            ],
                            vals,
                        )
                return 0

            lax.fori_loop(0, GROUPS // UNROLL, grp_body, 0)
            pltpu.sync_copy(obuf, out_hbm.at[pl.ds(t * OROWS, OROWS), :])

        return 0

    lax.fori_loop(0, ROUNDS, chunk_body, 0)


def kernel(z_vals, bin_edges, table):
    edges16 = jnp.pad(bin_edges[1:NUM_BINS], (1, 0))
    tablet = table.T.reshape(-1)

    mesh = plsc.VectorSubcoreMesh(core_axis_name="c", subcore_axis_name="s")
    out = pl.kernel(
        _body,
        out_type=jax.ShapeDtypeStruct((E_TOTAL * NUM_HEADS // 128, 128), jnp.float32),
        mesh=mesh,
        compiler_params=pltpu.CompilerParams(
            needs_layout_passes=False, use_tc_tiling_on_sc=True
        ),
        scratch_types=[
            pltpu.VMEM((CHUNK,), jnp.float32),
            pltpu.VMEM((OROWS, 128), jnp.float32),
            pltpu.VMEM((NUM_BINS * NUM_HEADS,), jnp.float32),
            pltpu.VMEM((L,), jnp.float32),
        ],
    )(z_vals, edges16, tablet)
    return out.reshape(E_TOTAL, NUM_HEADS)


# speedup vs baseline: 40.2633x; 5.0927x over previous
import jax
import jax.numpy as jnp
from jax import lax
from jax.experimental import pallas as pl
from jax.experimental.pallas import tpu as pltpu
from jax.experimental.pallas import tpu_sc as plsc

NUM_HEADS = 8
NUM_BINS = 16
N_EDGES_INNER = NUM_BINS - 1
E_TOTAL = 3200000
NC, NS, L = 2, 16, 16
NW = NC * NS
CHUNK = 3200
NCHUNKS = E_TOTAL // CHUNK
ROUNDS = -(-NCHUNKS // NW)
TILES = CHUNK // 128
GPT = 128 // L


def _body(z_hbm, edges_hbm, tablet_hbm, out_hbm, zbuf, obuf, table_v, edges_v):
    wid = lax.axis_index("s") * NC + lax.axis_index("c")

    pltpu.sync_copy(tablet_hbm, table_v)
    pltpu.sync_copy(edges_hbm, edges_v)

    edges = [
        plsc.load_gather(edges_v, [jnp.full((L,), i + 1, jnp.int32)])
        for i in range(N_EDGES_INNER)
    ]
    gpat = [jnp.full((L,), h * NUM_BINS, jnp.int32) for h in range(NUM_HEADS)]

    def chunk_body(r, _):
        t = r * NW + wid

        @pl.when(t < NCHUNKS)
        def _():
            zoff = t * CHUNK
            pltpu.sync_copy(z_hbm.at[pl.ds(zoff, CHUNK)], zbuf)

            def tile_body(tt, _):
                for u in range(GPT):
                    off = tt * 128 + u * L
                    z = zbuf[pl.ds(off, L)]
                    bin_idx = jnp.zeros((L,), jnp.int32)
                    one = jnp.ones((L,), jnp.int32)
                    zero = jnp.zeros((L,), jnp.int32)
                    for e in edges:
                        bin_idx += jnp.where(z > e, one, zero)
                    for h in range(NUM_HEADS):
                        vals = plsc.load_gather(table_v, [bin_idx + gpat[h]])
                        obuf[tt, h, pl.ds(u * L, L)] = vals
                return 0

            lax.fori_loop(0, TILES, tile_body, 0)
            pltpu.sync_copy(obuf, out_hbm.at[pl.ds(t * TILES, TILES)])

        return 0

    lax.fori_loop(0, ROUNDS, chunk_body, 0)


def kernel(z_vals, bin_edges, table):
    edges16 = jnp.pad(bin_edges[1:NUM_BINS], (1, 0))
    tablet = table.T.reshape(-1)

    mesh = plsc.VectorSubcoreMesh(core_axis_name="c", subcore_axis_name="s")
    out3 = pl.kernel(
        _body,
        out_type=jax.ShapeDtypeStruct((E_TOTAL // 128, NUM_HEADS, 128), jnp.float32),
        mesh=mesh,
        compiler_params=pltpu.CompilerParams(needs_layout_passes=False),
        scratch_types=[
            pltpu.VMEM((CHUNK,), jnp.float32),
            pltpu.VMEM((TILES, NUM_HEADS, 128), jnp.float32),
            pltpu.VMEM((NUM_BINS * NUM_HEADS,), jnp.float32),
            pltpu.VMEM((L,), jnp.float32),
        ],
    )(z_vals, edges16, tablet)
    return out3.transpose(0, 2, 1).reshape(E_TOTAL, NUM_HEADS)


# speedup vs baseline: 47.8089x; 1.1874x over previous
import jax
import jax.numpy as jnp
from jax import lax
from jax.experimental import pallas as pl
from jax.experimental.pallas import tpu as pltpu
from jax.experimental.pallas import tpu_sc as plsc

NUM_HEADS = 8
NUM_BINS = 16
N_EDGES_INNER = NUM_BINS - 1
E_TOTAL = 3200000
NC, NS, L = 2, 16, 16
NW = NC * NS
CHUNK = 3200
NCHUNKS = E_TOTAL // CHUNK
ROUNDS = -(-NCHUNKS // NW)
TILES = CHUNK // 128
GPT = 128 // L


def _body(z_hbm, edges_hbm, tablet_hbm, out_hbm, zbuf, obuf, table_v, edges_v,
          sin0, sin1, sout0, sout1):
    wid = lax.axis_index("s") * NC + lax.axis_index("c")

    pltpu.sync_copy(tablet_hbm, table_v)
    pltpu.sync_copy(edges_hbm, edges_v)

    edges = [
        plsc.load_gather(edges_v, [jnp.full((L,), i + 1, jnp.int32)])
        for i in range(N_EDGES_INNER)
    ]
    gpat = [jnp.full((L,), h * NUM_BINS, jnp.int32) for h in range(NUM_HEADS)]
    sin = (sin0, sin1)
    sout = (sout0, sout1)

    def start_in(t, s):
        @pl.when(t < NCHUNKS)
        def _():
            pltpu.async_copy(z_hbm.at[pl.ds(t * CHUNK, CHUNK)], zbuf.at[s], sin[s])

    def wait_in(t, s):
        @pl.when(t < NCHUNKS)
        def _():
            pltpu.make_async_copy(
                z_hbm.at[pl.ds(t * CHUNK, CHUNK)], zbuf.at[s], sin[s]
            ).wait()

    def start_out(t, s):
        @pl.when(t < NCHUNKS)
        def _():
            pltpu.async_copy(obuf.at[s], out_hbm.at[pl.ds(t * TILES, TILES)], sout[s])

    def wait_out(t, s):
        @pl.when((t >= 0) & (t < NCHUNKS))
        def _():
            pltpu.make_async_copy(
                obuf.at[s], out_hbm.at[pl.ds(t * TILES, TILES)], sout[s]
            ).wait()

    def compute(t, s):
        @pl.when(t < NCHUNKS)
        def _():
            def tile_body(tt, _):
                for u in range(GPT):
                    off = tt * 128 + u * L
                    z = zbuf[s, pl.ds(off, L)]
                    bin_idx = jnp.zeros((L,), jnp.int32)
                    one = jnp.ones((L,), jnp.int32)
                    zero = jnp.zeros((L,), jnp.int32)
                    for e in edges:
                        bin_idx += jnp.where(z > e, one, zero)
                    for h in range(NUM_HEADS):
                        vals = plsc.load_gather(table_v, [bin_idx + gpat[h]])
                        obuf[s, tt, h, pl.ds(u * L, L)] = vals
                return 0

            lax.fori_loop(0, TILES, tile_body, 0)

    start_in(wid, 0)

    def pair_body(k, _):
        tA = 2 * k * NW + wid
        tB = tA + NW
        tC = tB + NW
        start_in(tB, 1)
        wait_in(tA, 0)
        wait_out(tA - 2 * NW, 0)
        compute(tA, 0)
        start_out(tA, 0)
        start_in(tC, 0)
        wait_in(tB, 1)
        wait_out(tB - 2 * NW, 1)
        compute(tB, 1)
        start_out(tB, 1)
        return 0

    lax.fori_loop(0, ROUNDS // 2, pair_body, 0)
    wait_out((ROUNDS - 2) * NW + wid, 0)
    wait_out((ROUNDS - 1) * NW + wid, 1)


def kernel(z_vals, bin_edges, table):
    edges16 = jnp.pad(bin_edges[1:NUM_BINS], (1, 0))
    tablet = table.T.reshape(-1)

    mesh = plsc.VectorSubcoreMesh(core_axis_name="c", subcore_axis_name="s")
    out3 = pl.kernel(
        _body,
        out_type=jax.ShapeDtypeStruct((E_TOTAL // 128, NUM_HEADS, 128), jnp.float32),
        mesh=mesh,
        compiler_params=pltpu.CompilerParams(needs_layout_passes=False),
        scratch_types=[
            pltpu.VMEM((2, CHUNK), jnp.float32),
            pltpu.VMEM((2, TILES, NUM_HEADS, 128), jnp.float32),
            pltpu.VMEM((NUM_BINS * NUM_HEADS,), jnp.float32),
            pltpu.VMEM((L,), jnp.float32),
            pltpu.SemaphoreType.DMA,
            pltpu.SemaphoreType.DMA,
            pltpu.SemaphoreType.DMA,
            pltpu.SemaphoreType.DMA,
        ],
    )(z_vals, edges16, tablet)
    return out3.transpose(0, 2, 1).reshape(E_TOTAL, NUM_HEADS)
